# Initial kernel scaffold; baseline (speedup 1.0000x reference)
import jax
import jax.numpy as jnp
from jax.experimental import pallas as pl


def kernel(obs, edge_index, W1, b1, W2, b2, W3, b3):
    raise NotImplementedError("write your pallas kernel here")


# speedup vs baseline: 55.7852x; 55.7852x over previous
import functools
import jax
import jax.numpy as jnp
from jax import lax
from jax.experimental import pallas as pl
from jax.experimental.pallas import tpu as pltpu, tpu_sc as plsc

NC = 2
NS = 16
NW = NC * NS

f32 = jnp.float32
i32 = jnp.int32


def _mesh():
    return plsc.VectorSubcoreMesh(
        core_axis_name="c", subcore_axis_name="s", num_cores=NC, num_subcores=NS
    )


_SC_PARAMS = pltpu.CompilerParams(needs_layout_passes=False)


def _zero_1d(ref, n16, dtype):
    def b(i, _):
        ref[pl.ds(i * 16, 16)] = jnp.zeros((16,), dtype)
        return 0
    lax.fori_loop(0, n16, b, 0)


def _p1_hist_select(NP2, E, EW, N):
    NEV = EW // 16

    @functools.partial(
        pl.kernel,
        mesh=_mesh(),
        compiler_params=_SC_PARAMS,
        out_type=(
            jax.ShapeDtypeStruct((NW * NP2,), f32),
            jax.ShapeDtypeStruct((NW * EW,), i32),
            jax.ShapeDtypeStruct((NW * 16,), i32),
        ),
        scratch_types=[
            pltpu.VMEM((EW,), i32),
            pltpu.VMEM((EW,), i32),
            pltpu.VMEM((NP2,), f32),
            pltpu.VMEM((EW,), i32),
            pltpu.VMEM((16,), i32),
        ],
    )
    def k(src_hbm, dst_hbm, hist_hbm, sel_hbm, cnt_hbm,
          src_v, dst_v, hist_v, sel_v, cbuf):
        cid = lax.axis_index("c")
        sid = lax.axis_index("s")
        wid = cid * NS + sid
        pltpu.sync_copy(src_hbm.at[pl.ds(wid * EW, EW)], src_v)
        pltpu.sync_copy(dst_hbm.at[pl.ds(wid * EW, EW)], dst_v)
        _zero_1d(hist_v, NP2 // 16, f32)
        ones = jnp.ones((16,), f32)
        tvec = jnp.full((16,), N - 1, i32)

        def body(i, cnt):
            d = dst_v[pl.ds(i * 16, 16)]
            plsc.addupdate_scatter(hist_v, [d], ones)
            m = d == tvec
            pc = plsc.all_reduce_population_count(m)[0]

            def hit(c):
                s_ = src_v[pl.ds(i * 16, 16)]
                mi = m.astype(i32)
                pos = c + plsc.cumsum(mi) - mi
                plsc.store_scatter(sel_v, [pos], s_, mask=m)
                return c + pc
            return lax.cond(pc > 0, hit, lambda c: c, cnt)

        cnt = lax.fori_loop(0, NEV, body, jnp.asarray(0, i32))
        cbuf[...] = jnp.full((16,), cnt, i32)
        pltpu.sync_copy(cbuf, cnt_hbm.at[pl.ds(wid * 16, 16)])

        def selw(kk, _):
            pltpu.sync_copy(sel_v.at[pl.ds(kk * 16, 16)],
                            sel_hbm.at[pl.ds(wid * EW + kk * 16, 16)])
            return 0
        lax.fori_loop(0, (cnt + 15) // 16, selw, 0)

        pltpu.sync_copy(hist_v, hist_hbm.at[pl.ds(wid * NP2, NP2)])

    return k


def _p3a_flag(NP2, EW, N):
    @functools.partial(
        pl.kernel,
        mesh=_mesh(),
        compiler_params=_SC_PARAMS,
        out_type=jax.ShapeDtypeStruct((NP2,), i32),
        scratch_types=[
            pltpu.VMEM((NP2,), i32),
            pltpu.VMEM((16,), i32),
            pltpu.VMEM((16,), i32),
        ],
    )
    def k(sel_hbm, cnt_hbm, flag_hbm, flag_v, cbuf, selb):
        cid = lax.axis_index("c")
        sid = lax.axis_index("s")

        @pl.when(jnp.logical_and(cid == 0, sid == 0))
        def _():
            lane = lax.iota(i32, 16)
            onesi = jnp.ones((16,), i32)
            _zero_1d(flag_v, NP2 // 16, i32)
            plsc.store_scatter(flag_v, [jnp.full((16,), N - 1, i32)], onesi,
                               mask=lane == 0)
            for w in range(NW):
                pltpu.sync_copy(cnt_hbm.at[pl.ds(w * 16, 16)], cbuf)
                cw = cbuf[...][0]

                def sb(kk, _, w=w):
                    pltpu.sync_copy(sel_hbm.at[pl.ds(w * EW + kk * 16, 16)],
                                    selb)
                    vec = selb[...]
                    m = lane < (cw - kk * 16)
                    vecc = jnp.where(m, vec, 0)
                    plsc.store_scatter(flag_v, [vecc], onesi, mask=m)
                    return 0
                lax.fori_loop(0, (cw + 15) // 16, sb, 0)

            pltpu.sync_copy(flag_v, flag_hbm)

    return k


def _p2_dinv_comp(NP2):
    R = NP2 // 128

    def body(hist_ref, flag_ref, dinv_ref, comp_ref, s_ref):
        deg = jnp.sum(hist_ref[...], axis=0) + 1.0
        dinv_ref[...] = lax.rsqrt(deg)
        fl = flag_ref[...].astype(f32)
        rk = lax.broadcasted_iota(i32, (128, 128), 0)
        ck = lax.broadcasted_iota(i32, (128, 128), 1)
        lt = (rk <= ck).astype(f32)
        rowcum = jnp.dot(fl, lt, preferred_element_type=f32,
                         precision=lax.Precision.HIGHEST)
        rq = lax.broadcasted_iota(i32, (R, R), 0)
        cq = lax.broadcasted_iota(i32, (R, R), 1)
        st = (cq < rq).astype(f32)
        rowtot = rowcum[:, 127:128]
        rowbase = jnp.dot(st, rowtot, preferred_element_type=f32,
                          precision=lax.Precision.HIGHEST)
        comp_ref[...] = (rowbase + rowcum - fl).astype(i32)
        s_ref[...] = jnp.full((8, 128), jnp.sum(fl), f32).astype(i32)

    return pl.pallas_call(
        body,
        out_shape=(
            jax.ShapeDtypeStruct((R, 128), f32),
            jax.ShapeDtypeStruct((R, 128), i32),
            jax.ShapeDtypeStruct((8, 128), i32),
        ),
    )


def _p4_aggregate(NP2, E, EW, N, DIN):
    HW = 2560
    WIN = NC * HW
    SWZ = HW // NS
    SW2 = NP2 // NS
    ZR = 64
    CV = DIN // 16
    CAP = 2 * EW
    NEV2 = CAP // 16

    @functools.partial(
        pl.kernel,
        mesh=_mesh(),
        compiler_params=_SC_PARAMS,
        out_type=jax.ShapeDtypeStruct((NP2, DIN), f32),
        scratch_types=[
            pltpu.VMEM((CAP,), i32),
            pltpu.VMEM((CAP,), i32),
            pltpu.VMEM((NP2,), f32),
            pltpu.VMEM((NP2,), i32),
            pltpu.VMEM((NP2,), i32),
            pltpu.VMEM((CAP,), i32),
            pltpu.VMEM((16, DIN), f32),
            pltpu.VMEM((ZR, DIN), f32),
            pltpu.VMEM((16,), i32),
            pltpu.VMEM((16,), i32),
            pltpu.VMEM((16,), i32),
            pltpu.VMEM_SHARED((HW, DIN), f32),
            pltpu.SemaphoreType.DMA,
        ],
    )
    def k(src_hbm, dst_hbm, obs_hbm, dinv_hbm, flag_hbm, comp_hbm,
          s_hbm, zer_hbm, a_hbm,
          src_v, dst_v, dinv_v, flag_v, comp_v, beidx_v,
          rows_v, zrows_v, sbuf, gibuf, sibuf, acc_sh, sem):
        cid = lax.axis_index("c")
        sid = lax.axis_index("s")
        lane = lax.iota(i32, 16)

        pltpu.sync_copy(zer_hbm, zrows_v)
        pltpu.sync_copy(dinv_hbm, dinv_v)
        pltpu.sync_copy(flag_hbm, flag_v)
        pltpu.sync_copy(comp_hbm, comp_v)
        pltpu.sync_copy(s_hbm, sbuf)
        pltpu.sync_copy(src_hbm.at[pl.ds(sid * EW, EW)],
                        src_v.at[pl.ds(0, EW)])
        pltpu.sync_copy(src_hbm.at[pl.ds((NS + sid) * EW, EW)],
                        src_v.at[pl.ds(EW, EW)])
        pltpu.sync_copy(dst_hbm.at[pl.ds(sid * EW, EW)],
                        dst_v.at[pl.ds(0, EW)])
        pltpu.sync_copy(dst_hbm.at[pl.ds((NS + sid) * EW, EW)],
                        dst_v.at[pl.ds(EW, EW)])
        s = sbuf[...][0]

        def scan(i, bcnt):
            d = dst_v[pl.ds(i * 16, 16)]
            fl = plsc.load_gather(flag_v, [d])
            m = fl > 0
            pc = plsc.all_reduce_population_count(m)[0]

            def hit(c):
                mi = m.astype(i32)
                pos = c + plsc.cumsum(mi) - mi
                plsc.store_scatter(beidx_v, [pos], lane + i * 16, mask=m)
                return c + pc
            return lax.cond(pc > 0, hit, lambda c: c, bcnt)

        bcnt = lax.fori_loop(0, NEV2, scan, jnp.asarray(0, i32))
        plsc.subcore_barrier()

        nrounds = (s + WIN - 1) // WIN

        def round_body(r, _):
            lo = r * WIN + cid * HW

            def zr(kk, __):
                pltpu.sync_copy(zrows_v,
                                acc_sh.at[pl.ds(sid * SWZ + kk * ZR, ZR)])
                return 0
            lax.fori_loop(0, SWZ // ZR, zr, 0)
            plsc.subcore_barrier()

            def process_chunk(svec, w, cvec, valid):
                valid = jnp.logical_and(
                    valid, jnp.logical_and(cvec >= lo, cvec < lo + HW))
                w = jnp.where(valid, w, 0.0)
                gibuf[...] = svec
                sibuf[...] = jnp.where(valid, cvec - lo, 0)
                pltpu.async_copy(obs_hbm.at[gibuf], rows_v, sem).wait()
                for l in range(16):
                    wv = jnp.full((16,), w[l])
                    for c in range(CV):
                        rows_v[l, pl.ds(c * 16, 16)] = (
                            rows_v[l, pl.ds(c * 16, 16)] * wv
                        )
                pltpu.sync_copy(rows_v, acc_sh.at[sibuf], add=True)

            def flush(j, __):
                off = j * 16
                ev = beidx_v[pl.ds(off, 16)]
                valid = lane < (bcnt - off)
                ev = jnp.where(valid, ev, 0)
                sv = plsc.load_gather(src_v, [ev])
                dv = plsc.load_gather(dst_v, [ev])
                sv = jnp.where(valid, sv, 0)
                dv = jnp.where(valid, dv, 0)
                w = (plsc.load_gather(dinv_v, [sv])
                     * plsc.load_gather(dinv_v, [dv]))
                cvec = plsc.load_gather(comp_v, [dv])
                process_chunk(sv, w, cvec, valid)
                return 0
            lax.fori_loop(0, (bcnt + 15) // 16, flush, 0)

            def selfb(j, __):
                vv = lane + sid * SW2 + j * 16
                f = flag_v[pl.ds(sid * SW2 + j * 16, 16)]
                m = f > 0
                pc = plsc.all_reduce_population_count(m)[0]

                def hit(u):
                    dv_ = plsc.load_gather(dinv_v, [vv])
                    w = dv_ * dv_
                    cvec = plsc.load_gather(comp_v, [vv])
                    process_chunk(vv, w, cvec, m)
                    return 0
                return lax.cond(pc > 0, hit, lambda u: 0, 0)
            lax.fori_loop(0, SW2 // 16, selfb, 0)

            plsc.subcore_barrier()

            @pl.when(sid == 0)
            def _():
                sc = jnp.clip(s - lo, 0, HW)

                def outw(kk, __):
                    pltpu.sync_copy(acc_sh.at[pl.ds(kk * ZR, ZR)],
                                    a_hbm.at[pl.ds(lo + kk * ZR, ZR)])
                    return 0
                lax.fori_loop(0, (sc + ZR - 1) // ZR, outw, 0)
            plsc.subcore_barrier()
            return 0

        lax.fori_loop(0, nrounds, round_body, 0)

    return k


def _p5_matmul(NP2, DIN, H):
    BR = 256

    def body(s_ref, a_ref, w1_ref, b1_ref, h_ref, av, hv, sem0, sem1):
        s = s_ref[0]
        nb = (s + BR - 1) // BR

        def blk(kk, _):
            cp0 = pltpu.make_async_copy(a_ref.at[pl.ds(kk * BR, BR)], av, sem0)
            cp0.start()
            cp0.wait()
            h = jnp.maximum(
                jnp.dot(av[...], w1_ref[...], preferred_element_type=f32,
                        precision=lax.Precision.HIGHEST)
                + b1_ref[...], 0.0)
            hv[...] = jnp.concatenate([h, jnp.zeros_like(h)], axis=1)
            cph = pltpu.make_async_copy(
                hv, h_ref.at[pl.ds(kk * BR, BR)], sem1)
            cph.start()
            cph.wait()
            return 0
        lax.fori_loop(0, nb, blk, 0)

    return pl.pallas_call(
        body,
        in_specs=[
            pl.BlockSpec(memory_space=pltpu.SMEM),
            pl.BlockSpec(memory_space=pltpu.HBM),
            pl.BlockSpec(memory_space=pltpu.VMEM),
            pl.BlockSpec(memory_space=pltpu.VMEM),
        ],
        out_specs=pl.BlockSpec(memory_space=pltpu.HBM),
        out_shape=jax.ShapeDtypeStruct((NP2, 2 * H), f32),
        scratch_shapes=[
            pltpu.VMEM((BR, DIN), f32),
            pltpu.VMEM((BR, 2 * H), f32),
            pltpu.SemaphoreType.DMA,
            pltpu.SemaphoreType.DMA,
        ],
    )


def _p6_layer2(NP2, EW, N, H):
    CV = H // 16

    @functools.partial(
        pl.kernel,
        mesh=_mesh(),
        compiler_params=_SC_PARAMS,
        out_type=jax.ShapeDtypeStruct((NC * H,), f32),
        scratch_types=[
            pltpu.VMEM((NP2,), f32),
            pltpu.VMEM((NP2,), i32),
            pltpu.VMEM((16,), i32),
            pltpu.VMEM((16,), i32),
            pltpu.VMEM((16,), i32),
            pltpu.VMEM((16, 2 * H), f32),
            pltpu.VMEM((H,), f32),
            pltpu.VMEM((NS * H,), f32),
            pltpu.VMEM_SHARED((NS * H,), f32),
            pltpu.SemaphoreType.DMA,
        ],
    )
    def k(h_hbm, sel_hbm, cnt_hbm, dinv_hbm, comp_hbm, zpart_hbm,
          dinv_v, comp_v, cbuf, selb, gibuf, hrows_v, zacc_v, tmpall_v,
          zslots, sem):
        cid = lax.axis_index("c")
        sid = lax.axis_index("s")
        wid = cid * NS + sid
        lane = lax.iota(i32, 16)
        pltpu.sync_copy(cnt_hbm.at[pl.ds(wid * 16, 16)], cbuf)
        cw = cbuf[...][0]
        _zero_1d(zacc_v, CV, f32)

        @pl.when(jnp.logical_or(cw > 0, wid == 0))
        def _():
            pltpu.sync_copy(dinv_hbm, dinv_v)
            pltpu.sync_copy(comp_hbm, comp_v)
            dinvt = dinv_v[pl.ds(N - 16, 16)][15]

            def accum(w, rows16):
                for l in range(16):
                    wv = jnp.full((16,), w[l])
                    for c in range(CV):
                        zacc_v[pl.ds(c * 16, 16)] = (
                            zacc_v[pl.ds(c * 16, 16)]
                            + rows16[l, pl.ds(c * 16, 16)] * wv
                        )

            def chunk(j, _):
                pltpu.sync_copy(sel_hbm.at[pl.ds(wid * EW + j * 16, 16)],
                                selb)
                vv = selb[...]
                valid = lane < (cw - j * 16)
                vv = jnp.where(valid, vv, 0)
                w = jnp.where(valid,
                              plsc.load_gather(dinv_v, [vv]) * dinvt, 0.0)
                gibuf[...] = jnp.where(valid,
                                       plsc.load_gather(comp_v, [vv]), 0)
                pltpu.async_copy(h_hbm.at[gibuf], hrows_v, sem).wait()
                accum(w, hrows_v)
                return 0
            lax.fori_loop(0, (cw + 15) // 16, chunk, 0)

            @pl.when(wid == 0)
            def _():
                gibuf[...] = jnp.full((16,), comp_v[pl.ds(N - 16, 16)][15])
                pltpu.async_copy(h_hbm.at[gibuf], hrows_v, sem).wait()
                w = jnp.where(lane == 0, dinvt * dinvt, 0.0)
                accum(w, hrows_v)

        pltpu.sync_copy(zacc_v, zslots.at[pl.ds(sid * H, H)])
        plsc.subcore_barrier()

        @pl.when(sid == 0)
        def _():
            pltpu.sync_copy(zslots, tmpall_v)
            _zero_1d(zacc_v, CV, f32)
            for j in range(NS):
                for c in range(CV):
                    zacc_v[pl.ds(c * 16, 16)] = (
                        zacc_v[pl.ds(c * 16, 16)]
                        + tmpall_v[pl.ds(j * H + c * 16, 16)]
                    )
            pltpu.sync_copy(zacc_v, zpart_hbm.at[pl.ds(cid * H, H)])

    return k


def _p7_head(H, DOUT):
    def body(zp_ref, w2_ref, b2_ref, w3_ref, b3_ref, out_ref):
        z = zp_ref[0:1, :] + zp_ref[1:2, :]
        z8 = jnp.broadcast_to(z, (8, z.shape[1]))
        h2 = jnp.maximum(
            jnp.dot(z8, w2_ref[...], preferred_element_type=f32,
                    precision=lax.Precision.HIGHEST) + b2_ref[...],
            0.0)
        out_ref[...] = (
            jnp.dot(h2, w3_ref[...], preferred_element_type=f32,
                    precision=lax.Precision.HIGHEST) + b3_ref[...]
        )

    return pl.pallas_call(
        body,
        out_shape=jax.ShapeDtypeStruct((8, DOUT), f32),
    )


def kernel(obs, edge_index, W1, b1, W2, b2, W3, b3):
    N, DIN = obs.shape
    E = edge_index.shape[1]
    H = W1.shape[1]
    DOUT = W3.shape[1]
    assert E % NW == 0
    EW = E // NW
    assert EW % 16 == 0
    NP2 = ((N + 1279) // 1280) * 1280
    zer = jnp.zeros((64, DIN), f32)

    esrc = edge_index[0]
    edst = edge_index[1]
    hist, sel, cnt = _p1_hist_select(NP2, E, EW, N)(esrc, edst)
    flag = _p3a_flag(NP2, EW, N)(sel, cnt)
    dinv, comp, s = _p2_dinv_comp(NP2)(hist.reshape(NW, NP2 // 128, 128),
                                       flag.reshape(NP2 // 128, 128))
    dinv = dinv.reshape(NP2)
    comp = comp.reshape(NP2)
    s16 = s.reshape(-1)[:16]
    a = _p4_aggregate(NP2, E, EW, N, DIN)(
        esrc, edst, obs, dinv, flag, comp, s16, zer)
    h = _p5_matmul(NP2, DIN, H)(s16, a, W1, b1.reshape(1, H))
    zpart = _p6_layer2(NP2, EW, N, H)(h, sel, cnt, dinv, comp)
    out = _p7_head(H, DOUT)(zpart.reshape(NC, H), W2, b2.reshape(1, H), W3,
                            b3.reshape(1, DOUT))
    return out[0]


# speedup vs baseline: 68.6716x; 1.2310x over previous
import functools
import jax
import jax.numpy as jnp
from jax import lax
from jax.experimental import pallas as pl
from jax.experimental.pallas import tpu as pltpu, tpu_sc as plsc

NC = 2
NS = 16
NW = NC * NS

f32 = jnp.float32
i32 = jnp.int32


def _mesh():
    return plsc.VectorSubcoreMesh(
        core_axis_name="c", subcore_axis_name="s", num_cores=NC, num_subcores=NS
    )


_SC_PARAMS = pltpu.CompilerParams(needs_layout_passes=False)


def _zero_1d(ref, n16, dtype):
    def b(i, _):
        ref[pl.ds(i * 16, 16)] = jnp.zeros((16,), dtype)
        return 0
    lax.fori_loop(0, n16, b, 0)


def _p1_hist_select(NP2, E, EW, N):
    NEV = EW // 16

    @functools.partial(
        pl.kernel,
        mesh=_mesh(),
        compiler_params=_SC_PARAMS,
        out_type=(
            jax.ShapeDtypeStruct((NW * NP2,), f32),
            jax.ShapeDtypeStruct((NW * EW,), i32),
            jax.ShapeDtypeStruct((NW * 16,), i32),
        ),
        scratch_types=[
            pltpu.VMEM((EW,), i32),
            pltpu.VMEM((EW,), i32),
            pltpu.VMEM((NP2,), f32),
            pltpu.VMEM((EW,), i32),
            pltpu.VMEM((16,), i32),
        ],
    )
    def k(src_hbm, dst_hbm, hist_hbm, sel_hbm, cnt_hbm,
          src_v, dst_v, hist_v, sel_v, cbuf):
        cid = lax.axis_index("c")
        sid = lax.axis_index("s")
        wid = cid * NS + sid
        pltpu.sync_copy(src_hbm.at[pl.ds(wid * EW, EW)], src_v)
        pltpu.sync_copy(dst_hbm.at[pl.ds(wid * EW, EW)], dst_v)
        _zero_1d(hist_v, NP2 // 16, f32)
        ones = jnp.ones((16,), f32)
        tvec = jnp.full((16,), N - 1, i32)

        def body(i, cnt):
            d = dst_v[pl.ds(i * 16, 16)]
            plsc.addupdate_scatter(hist_v, [d], ones)
            m = d == tvec
            s_ = src_v[pl.ds(i * 16, 16)]
            mi = m.astype(i32)
            pos = cnt + plsc.cumsum(mi) - mi
            plsc.store_scatter(sel_v, [pos], s_, mask=m)
            return cnt + plsc.all_reduce_population_count(m)[0]

        cnt = lax.fori_loop(0, NEV, body, jnp.asarray(0, i32))
        cbuf[...] = jnp.full((16,), cnt, i32)
        pltpu.sync_copy(cbuf, cnt_hbm.at[pl.ds(wid * 16, 16)])

        def selw(kk, _):
            pltpu.sync_copy(sel_v.at[pl.ds(kk * 16, 16)],
                            sel_hbm.at[pl.ds(wid * EW + kk * 16, 16)])
            return 0
        lax.fori_loop(0, (cnt + 15) // 16, selw, 0)

        pltpu.sync_copy(hist_v, hist_hbm.at[pl.ds(wid * NP2, NP2)])

    return k


def _p3a_flag(NP2, EW, N):
    @functools.partial(
        pl.kernel,
        mesh=_mesh(),
        compiler_params=_SC_PARAMS,
        out_type=jax.ShapeDtypeStruct((NP2,), i32),
        scratch_types=[
            pltpu.VMEM((NP2,), i32),
            pltpu.VMEM((NW * 16,), i32),
            pltpu.VMEM((16,), i32),
        ],
    )
    def k(sel_hbm, cnt_hbm, flag_hbm, flag_v, cball, selb):
        cid = lax.axis_index("c")
        sid = lax.axis_index("s")

        @pl.when(jnp.logical_and(cid == 0, sid == 0))
        def _():
            lane = lax.iota(i32, 16)
            onesi = jnp.ones((16,), i32)
            _zero_1d(flag_v, NP2 // 16, i32)
            plsc.store_scatter(flag_v, [jnp.full((16,), N - 1, i32)], onesi,
                               mask=lane == 0)
            pltpu.sync_copy(cnt_hbm, cball)
            for w in range(NW):
                cw = cball[pl.ds(w * 16, 16)][0]

                def sb(kk, _, w=w):
                    pltpu.sync_copy(sel_hbm.at[pl.ds(w * EW + kk * 16, 16)],
                                    selb)
                    vec = selb[...]
                    m = lane < (cw - kk * 16)
                    vecc = jnp.where(m, vec, 0)
                    plsc.store_scatter(flag_v, [vecc], onesi, mask=m)
                    return 0
                lax.fori_loop(0, (cw + 15) // 16, sb, 0)

            pltpu.sync_copy(flag_v, flag_hbm)

    return k


def _p2_dinv_comp(NP2):
    R = NP2 // 128

    def body(hist_ref, flag_ref, dinv_ref, comp_ref, s_ref):
        deg = jnp.sum(hist_ref[...], axis=0) + 1.0
        dinv_ref[...] = lax.rsqrt(deg)
        fl = flag_ref[...].astype(f32)
        rk = lax.broadcasted_iota(i32, (128, 128), 0)
        ck = lax.broadcasted_iota(i32, (128, 128), 1)
        lt = (rk <= ck).astype(f32)
        rowcum = jnp.dot(fl, lt, preferred_element_type=f32,
                         precision=lax.Precision.HIGHEST)
        rq = lax.broadcasted_iota(i32, (R, R), 0)
        cq = lax.broadcasted_iota(i32, (R, R), 1)
        st = (cq < rq).astype(f32)
        rowtot = rowcum[:, 127:128]
        rowbase = jnp.dot(st, rowtot, preferred_element_type=f32,
                          precision=lax.Precision.HIGHEST)
        comp_ref[...] = (rowbase + rowcum - fl).astype(i32)
        s_ref[...] = jnp.full((8, 128), jnp.sum(fl), f32).astype(i32)

    return pl.pallas_call(
        body,
        out_shape=(
            jax.ShapeDtypeStruct((R, 128), f32),
            jax.ShapeDtypeStruct((R, 128), i32),
            jax.ShapeDtypeStruct((8, 128), i32),
        ),
    )


def _p4_aggregate(NP2, E, EW, N, DIN):
    HW = 2560
    WIN = NC * HW
    SWZ = HW // NS
    SW2 = NP2 // NS
    ZR = 64
    CV = DIN // 16
    CAP = 2 * EW
    NEV2 = CAP // 16

    @functools.partial(
        pl.kernel,
        mesh=_mesh(),
        compiler_params=_SC_PARAMS,
        out_type=jax.ShapeDtypeStruct((NP2, DIN), f32),
        scratch_types=[
            pltpu.VMEM((CAP,), i32),
            pltpu.VMEM((CAP,), i32),
            pltpu.VMEM((NP2,), f32),
            pltpu.VMEM((NP2,), i32),
            pltpu.VMEM((NP2,), i32),
            pltpu.VMEM((CAP,), i32),
            pltpu.VMEM((16, DIN), f32),
            pltpu.VMEM((ZR, DIN), f32),
            pltpu.VMEM((16,), i32),
            pltpu.VMEM((16,), i32),
            pltpu.VMEM((16,), i32),
            pltpu.VMEM_SHARED((HW, DIN), f32),
            pltpu.SemaphoreType.DMA,
        ],
    )
    def k(src_hbm, dst_hbm, obs_hbm, dinv_hbm, flag_hbm, comp_hbm,
          s_hbm, zer_hbm, a_hbm,
          src_v, dst_v, dinv_v, flag_v, comp_v, beidx_v,
          rows_v, zrows_v, sbuf, gibuf, sibuf, acc_sh, sem):
        cid = lax.axis_index("c")
        sid = lax.axis_index("s")
        lane = lax.iota(i32, 16)

        pltpu.sync_copy(zer_hbm, zrows_v)
        pltpu.sync_copy(dinv_hbm, dinv_v)
        pltpu.sync_copy(flag_hbm, flag_v)
        pltpu.sync_copy(comp_hbm, comp_v)
        pltpu.sync_copy(s_hbm, sbuf)
        pltpu.sync_copy(src_hbm.at[pl.ds(sid * EW, EW)],
                        src_v.at[pl.ds(0, EW)])
        pltpu.sync_copy(src_hbm.at[pl.ds((NS + sid) * EW, EW)],
                        src_v.at[pl.ds(EW, EW)])
        pltpu.sync_copy(dst_hbm.at[pl.ds(sid * EW, EW)],
                        dst_v.at[pl.ds(0, EW)])
        pltpu.sync_copy(dst_hbm.at[pl.ds((NS + sid) * EW, EW)],
                        dst_v.at[pl.ds(EW, EW)])
        s = sbuf[...][0]

        def scan(i, bcnt):
            d = dst_v[pl.ds(i * 16, 16)]
            fl = plsc.load_gather(flag_v, [d])
            m = fl > 0
            mi = m.astype(i32)
            pos = bcnt + plsc.cumsum(mi) - mi
            plsc.store_scatter(beidx_v, [pos], lane + i * 16, mask=m)
            return bcnt + plsc.all_reduce_population_count(m)[0]

        bcnt = lax.fori_loop(0, NEV2, scan, jnp.asarray(0, i32))
        plsc.subcore_barrier()

        nrounds = (s + WIN - 1) // WIN

        def round_body(r, _):
            lo = r * WIN + cid * HW

            def zr(kk, __):
                pltpu.sync_copy(zrows_v.at[pl.ds(0, 32)],
                                acc_sh.at[pl.ds(sid * SWZ + kk * 32, 32)])
                return 0
            lax.fori_loop(0, SWZ // 32, zr, 0)
            plsc.subcore_barrier()

            def process_chunk(svec, w, cvec, valid):
                valid = jnp.logical_and(
                    valid, jnp.logical_and(cvec >= lo, cvec < lo + HW))
                w = jnp.where(valid, w, 0.0)
                gibuf[...] = svec
                sibuf[...] = jnp.where(valid, cvec - lo, 0)
                pltpu.async_copy(obs_hbm.at[gibuf], rows_v, sem).wait()
                for l in range(16):
                    wv = jnp.full((16,), w[l])
                    for c in range(CV):
                        rows_v[l, pl.ds(c * 16, 16)] = (
                            rows_v[l, pl.ds(c * 16, 16)] * wv
                        )
                pltpu.sync_copy(rows_v, acc_sh.at[sibuf], add=True)

            def flush(j, __):
                off = j * 16
                ev = beidx_v[pl.ds(off, 16)]
                valid = lane < (bcnt - off)
                ev = jnp.where(valid, ev, 0)
                sv = plsc.load_gather(src_v, [ev])
                dv = plsc.load_gather(dst_v, [ev])
                sv = jnp.where(valid, sv, 0)
                dv = jnp.where(valid, dv, 0)
                w = (plsc.load_gather(dinv_v, [sv])
                     * plsc.load_gather(dinv_v, [dv]))
                cvec = plsc.load_gather(comp_v, [dv])
                process_chunk(sv, w, cvec, valid)
                return 0
            lax.fori_loop(0, (bcnt + 15) // 16, flush, 0)

            def selfb(j, __):
                vv = lane + sid * SW2 + j * 16
                f = flag_v[pl.ds(sid * SW2 + j * 16, 16)]
                m = f > 0
                pc = plsc.all_reduce_population_count(m)[0]

                def hit(u):
                    dv_ = plsc.load_gather(dinv_v, [vv])
                    w = dv_ * dv_
                    cvec = plsc.load_gather(comp_v, [vv])
                    process_chunk(vv, w, cvec, m)
                    return 0
                return lax.cond(pc > 0, hit, lambda u: 0, 0)
            lax.fori_loop(0, SW2 // 16, selfb, 0)

            plsc.subcore_barrier()

            @pl.when(sid == 0)
            def _():
                sc = jnp.clip(s - lo, 0, HW)

                def outw(kk, __):
                    pltpu.sync_copy(acc_sh.at[pl.ds(kk * ZR, ZR)],
                                    a_hbm.at[pl.ds(lo + kk * ZR, ZR)])
                    return 0
                lax.fori_loop(0, (sc + ZR - 1) // ZR, outw, 0)
            plsc.subcore_barrier()
            return 0

        lax.fori_loop(0, nrounds, round_body, 0)

    return k


def _p5_matmul(NP2, DIN, H):
    BR = 256

    def body(s_ref, a_ref, w1_ref, b1_ref, h_ref, av, hv, sem0, sem1):
        s = s_ref[0]
        nb = (s + BR - 1) // BR

        def blk(kk, _):
            cp0 = pltpu.make_async_copy(a_ref.at[pl.ds(kk * BR, BR)], av, sem0)
            cp0.start()
            cp0.wait()
            h = jnp.maximum(
                jnp.dot(av[...], w1_ref[...], preferred_element_type=f32,
                        precision=lax.Precision.HIGHEST)
                + b1_ref[...], 0.0)
            hv[...] = jnp.concatenate([h, jnp.zeros_like(h)], axis=1)
            cph = pltpu.make_async_copy(
                hv, h_ref.at[pl.ds(kk * BR, BR)], sem1)
            cph.start()
            cph.wait()
            return 0
        lax.fori_loop(0, nb, blk, 0)

    return pl.pallas_call(
        body,
        in_specs=[
            pl.BlockSpec(memory_space=pltpu.SMEM),
            pl.BlockSpec(memory_space=pltpu.HBM),
            pl.BlockSpec(memory_space=pltpu.VMEM),
            pl.BlockSpec(memory_space=pltpu.VMEM),
        ],
        out_specs=pl.BlockSpec(memory_space=pltpu.HBM),
        out_shape=jax.ShapeDtypeStruct((NP2, 2 * H), f32),
        scratch_shapes=[
            pltpu.VMEM((BR, DIN), f32),
            pltpu.VMEM((BR, 2 * H), f32),
            pltpu.SemaphoreType.DMA,
            pltpu.SemaphoreType.DMA,
        ],
    )


def _p6_layer2(NP2, EW, N, H):
    CV = H // 16

    @functools.partial(
        pl.kernel,
        mesh=_mesh(),
        compiler_params=_SC_PARAMS,
        out_type=jax.ShapeDtypeStruct((NC * H,), f32),
        scratch_types=[
            pltpu.VMEM((NP2,), f32),
            pltpu.VMEM((NP2,), i32),
            pltpu.VMEM((16,), i32),
            pltpu.VMEM((16,), i32),
            pltpu.VMEM((16,), i32),
            pltpu.VMEM((16, 2 * H), f32),
            pltpu.VMEM((H,), f32),
            pltpu.VMEM((NS * H,), f32),
            pltpu.VMEM_SHARED((NS * H,), f32),
            pltpu.SemaphoreType.DMA,
        ],
    )
    def k(h_hbm, sel_hbm, cnt_hbm, dinv_hbm, comp_hbm, zpart_hbm,
          dinv_v, comp_v, cbuf, selb, gibuf, hrows_v, zacc_v, tmpall_v,
          zslots, sem):
        cid = lax.axis_index("c")
        sid = lax.axis_index("s")
        wid = cid * NS + sid
        lane = lax.iota(i32, 16)
        pltpu.sync_copy(cnt_hbm.at[pl.ds(wid * 16, 16)], cbuf)
        cw = cbuf[...][0]
        _zero_1d(zacc_v, CV, f32)

        @pl.when(jnp.logical_or(cw > 0, wid == 0))
        def _():
            pltpu.sync_copy(dinv_hbm, dinv_v)
            pltpu.sync_copy(comp_hbm, comp_v)
            dinvt = dinv_v[pl.ds(N - 16, 16)][15]

            def accum(w, rows16):
                for l in range(16):
                    wv = jnp.full((16,), w[l])
                    for c in range(CV):
                        zacc_v[pl.ds(c * 16, 16)] = (
                            zacc_v[pl.ds(c * 16, 16)]
                            + rows16[l, pl.ds(c * 16, 16)] * wv
                        )

            def chunk(j, _):
                pltpu.sync_copy(sel_hbm.at[pl.ds(wid * EW + j * 16, 16)],
                                selb)
                vv = selb[...]
                valid = lane < (cw - j * 16)
                vv = jnp.where(valid, vv, 0)
                w = jnp.where(valid,
                              plsc.load_gather(dinv_v, [vv]) * dinvt, 0.0)
                gibuf[...] = jnp.where(valid,
                                       plsc.load_gather(comp_v, [vv]), 0)
                pltpu.async_copy(h_hbm.at[gibuf], hrows_v, sem).wait()
                accum(w, hrows_v)
                return 0
            lax.fori_loop(0, (cw + 15) // 16, chunk, 0)

            @pl.when(wid == 0)
            def _():
                gibuf[...] = jnp.full((16,), comp_v[pl.ds(N - 16, 16)][15])
                pltpu.async_copy(h_hbm.at[gibuf], hrows_v, sem).wait()
                w = jnp.where(lane == 0, dinvt * dinvt, 0.0)
                accum(w, hrows_v)

        pltpu.sync_copy(zacc_v, zslots.at[pl.ds(sid * H, H)])
        plsc.subcore_barrier()

        @pl.when(sid == 0)
        def _():
            pltpu.sync_copy(zslots, tmpall_v)
            _zero_1d(zacc_v, CV, f32)
            for j in range(NS):
                for c in range(CV):
                    zacc_v[pl.ds(c * 16, 16)] = (
                        zacc_v[pl.ds(c * 16, 16)]
                        + tmpall_v[pl.ds(j * H + c * 16, 16)]
                    )
            pltpu.sync_copy(zacc_v, zpart_hbm.at[pl.ds(cid * H, H)])

    return k


def _p7_head(H, DOUT):
    def body(zp_ref, w2_ref, b2_ref, w3_ref, b3_ref, out_ref):
        z = zp_ref[0:1, :] + zp_ref[1:2, :]
        z8 = jnp.broadcast_to(z, (8, z.shape[1]))
        h2 = jnp.maximum(
            jnp.dot(z8, w2_ref[...], preferred_element_type=f32,
                    precision=lax.Precision.HIGHEST) + b2_ref[...],
            0.0)
        out_ref[...] = (
            jnp.dot(h2, w3_ref[...], preferred_element_type=f32,
                    precision=lax.Precision.HIGHEST) + b3_ref[...]
        )

    return pl.pallas_call(
        body,
        out_shape=jax.ShapeDtypeStruct((8, DOUT), f32),
    )


def kernel(obs, edge_index, W1, b1, W2, b2, W3, b3):
    N, DIN = obs.shape
    E = edge_index.shape[1]
    H = W1.shape[1]
    DOUT = W3.shape[1]
    assert E % NW == 0
    EW = E // NW
    assert EW % 16 == 0
    NP2 = ((N + 1279) // 1280) * 1280
    zer = jnp.zeros((64, DIN), f32)

    esrc = edge_index[0]
    edst = edge_index[1]
    hist, sel, cnt = _p1_hist_select(NP2, E, EW, N)(esrc, edst)
    flag = _p3a_flag(NP2, EW, N)(sel, cnt)
    dinv, comp, s = _p2_dinv_comp(NP2)(hist.reshape(NW, NP2 // 128, 128),
                                       flag.reshape(NP2 // 128, 128))
    dinv = dinv.reshape(NP2)
    comp = comp.reshape(NP2)
    s16 = s.reshape(-1)[:16]
    a = _p4_aggregate(NP2, E, EW, N, DIN)(
        esrc, edst, obs, dinv, flag, comp, s16, zer)
    h = _p5_matmul(NP2, DIN, H)(s16, a, W1, b1.reshape(1, H))
    zpart = _p6_layer2(NP2, EW, N, H)(h, sel, cnt, dinv, comp)
    out = _p7_head(H, DOUT)(zpart.reshape(NC, H), W2, b2.reshape(1, H), W3,
                            b3.reshape(1, DOUT))
    return out[0]


# speedup vs baseline: 69.7856x; 1.0162x over previous
import functools
import jax
import jax.numpy as jnp
from jax import lax
from jax.experimental import pallas as pl
from jax.experimental.pallas import tpu as pltpu, tpu_sc as plsc

NC = 2
NS = 16
NW = NC * NS

f32 = jnp.float32
i32 = jnp.int32


def _mesh():
    return plsc.VectorSubcoreMesh(
        core_axis_name="c", subcore_axis_name="s", num_cores=NC, num_subcores=NS
    )


_SC_PARAMS = pltpu.CompilerParams(needs_layout_passes=False)


def _zero_1d(ref, n16, dtype):
    def b(i, _):
        ref[pl.ds(i * 16, 16)] = jnp.zeros((16,), dtype)
        return 0
    lax.fori_loop(0, n16, b, 0)


def _p1_hist_select(NP2, E, EW, N):
    NEV = EW // 16

    @functools.partial(
        pl.kernel,
        mesh=_mesh(),
        compiler_params=_SC_PARAMS,
        out_type=(
            jax.ShapeDtypeStruct((NW * NP2,), f32),
            jax.ShapeDtypeStruct((NW * EW,), i32),
            jax.ShapeDtypeStruct((NW * 16,), i32),
        ),
        scratch_types=[
            pltpu.VMEM((EW,), i32),
            pltpu.VMEM((EW,), i32),
            pltpu.VMEM((NP2,), f32),
            pltpu.VMEM((EW,), i32),
            pltpu.VMEM((16,), i32),
        ],
    )
    def k(src_hbm, dst_hbm, hist_hbm, sel_hbm, cnt_hbm,
          src_v, dst_v, hist_v, sel_v, cbuf):
        cid = lax.axis_index("c")
        sid = lax.axis_index("s")
        wid = cid * NS + sid
        pltpu.sync_copy(src_hbm.at[pl.ds(wid * EW, EW)], src_v)
        pltpu.sync_copy(dst_hbm.at[pl.ds(wid * EW, EW)], dst_v)
        _zero_1d(hist_v, NP2 // 16, f32)
        ones = jnp.ones((16,), f32)
        tvec = jnp.full((16,), N - 1, i32)

        def body(i, cnt):
            for u in range(5):
                off = (i * 5 + u) * 16
                d = dst_v[pl.ds(off, 16)]
                plsc.addupdate_scatter(hist_v, [d], ones)
                m = d == tvec
                s_ = src_v[pl.ds(off, 16)]
                mi = m.astype(i32)
                pos = cnt + plsc.cumsum(mi) - mi
                plsc.store_scatter(sel_v, [pos], s_, mask=m)
                cnt = cnt + plsc.all_reduce_population_count(m)[0]
            return cnt

        assert NEV % 5 == 0
        cnt = lax.fori_loop(0, NEV // 5, body, jnp.asarray(0, i32))
        cbuf[...] = jnp.full((16,), cnt, i32)
        pltpu.sync_copy(cbuf, cnt_hbm.at[pl.ds(wid * 16, 16)])

        def selw(kk, _):
            pltpu.sync_copy(sel_v.at[pl.ds(kk * 16, 16)],
                            sel_hbm.at[pl.ds(wid * EW + kk * 16, 16)])
            return 0
        lax.fori_loop(0, (cnt + 15) // 16, selw, 0)

        pltpu.sync_copy(hist_v, hist_hbm.at[pl.ds(wid * NP2, NP2)])

    return k


def _p3a_flag(NP2, EW, N):
    @functools.partial(
        pl.kernel,
        mesh=_mesh(),
        compiler_params=_SC_PARAMS,
        out_type=jax.ShapeDtypeStruct((NP2,), i32),
        scratch_types=[
            pltpu.VMEM((NP2,), i32),
            pltpu.VMEM((NW * 16,), i32),
            pltpu.VMEM((16,), i32),
        ],
    )
    def k(sel_hbm, cnt_hbm, flag_hbm, flag_v, cball, selb):
        cid = lax.axis_index("c")
        sid = lax.axis_index("s")

        @pl.when(jnp.logical_and(cid == 0, sid == 0))
        def _():
            lane = lax.iota(i32, 16)
            onesi = jnp.ones((16,), i32)
            _zero_1d(flag_v, NP2 // 16, i32)
            plsc.store_scatter(flag_v, [jnp.full((16,), N - 1, i32)], onesi,
                               mask=lane == 0)
            pltpu.sync_copy(cnt_hbm, cball)
            for w in range(NW):
                cw = cball[pl.ds(w * 16, 16)][0]

                def sb(kk, _, w=w):
                    pltpu.sync_copy(sel_hbm.at[pl.ds(w * EW + kk * 16, 16)],
                                    selb)
                    vec = selb[...]
                    m = lane < (cw - kk * 16)
                    vecc = jnp.where(m, vec, 0)
                    plsc.store_scatter(flag_v, [vecc], onesi, mask=m)
                    return 0
                lax.fori_loop(0, (cw + 15) // 16, sb, 0)

            pltpu.sync_copy(flag_v, flag_hbm)

    return k


def _p2_dinv_comp(NP2):
    R = NP2 // 128

    def body(hist_ref, flag_ref, dinv_ref, comp_ref, s_ref):
        deg = jnp.sum(hist_ref[...], axis=0) + 1.0
        dinv_ref[...] = lax.rsqrt(deg)
        fl = flag_ref[...].astype(f32)
        rk = lax.broadcasted_iota(i32, (128, 128), 0)
        ck = lax.broadcasted_iota(i32, (128, 128), 1)
        lt = (rk <= ck).astype(f32)
        rowcum = jnp.dot(fl, lt, preferred_element_type=f32,
                         precision=lax.Precision.HIGHEST)
        rq = lax.broadcasted_iota(i32, (R, R), 0)
        cq = lax.broadcasted_iota(i32, (R, R), 1)
        st = (cq < rq).astype(f32)
        rowtot = rowcum[:, 127:128]
        rowbase = jnp.dot(st, rowtot, preferred_element_type=f32,
                          precision=lax.Precision.HIGHEST)
        comp_ref[...] = (rowbase + rowcum - fl).astype(i32)
        s_ref[...] = jnp.full((8, 128), jnp.sum(fl), f32).astype(i32)

    return pl.pallas_call(
        body,
        out_shape=(
            jax.ShapeDtypeStruct((R, 128), f32),
            jax.ShapeDtypeStruct((R, 128), i32),
            jax.ShapeDtypeStruct((8, 128), i32),
        ),
    )


def _p4_aggregate(NP2, E, EW, N, DIN):
    HW = 2560
    WIN = NC * HW
    SWZ = HW // NS
    SW2 = NP2 // NS
    ZR = 64
    CV = DIN // 16
    CAP = 2 * EW
    NEV2 = CAP // 16

    @functools.partial(
        pl.kernel,
        mesh=_mesh(),
        compiler_params=_SC_PARAMS,
        out_type=jax.ShapeDtypeStruct((NP2, DIN), f32),
        scratch_types=[
            pltpu.VMEM((CAP,), i32),
            pltpu.VMEM((CAP,), i32),
            pltpu.VMEM((NP2,), f32),
            pltpu.VMEM((NP2,), i32),
            pltpu.VMEM((NP2,), i32),
            pltpu.VMEM((CAP,), i32),
            pltpu.VMEM((16, DIN), f32),
            pltpu.VMEM((ZR, DIN), f32),
            pltpu.VMEM((16,), i32),
            pltpu.VMEM((16,), i32),
            pltpu.VMEM((16,), i32),
            pltpu.VMEM_SHARED((HW, DIN), f32),
            pltpu.SemaphoreType.DMA,
        ],
    )
    def k(src_hbm, dst_hbm, obs_hbm, dinv_hbm, flag_hbm, comp_hbm,
          s_hbm, zer_hbm, a_hbm,
          src_v, dst_v, dinv_v, flag_v, comp_v, beidx_v,
          rows_v, zrows_v, sbuf, gibuf, sibuf, acc_sh, sem):
        cid = lax.axis_index("c")
        sid = lax.axis_index("s")
        lane = lax.iota(i32, 16)

        pltpu.sync_copy(zer_hbm, zrows_v)
        pltpu.sync_copy(dinv_hbm, dinv_v)
        pltpu.sync_copy(flag_hbm, flag_v)
        pltpu.sync_copy(comp_hbm, comp_v)
        pltpu.sync_copy(s_hbm, sbuf)
        pltpu.sync_copy(src_hbm.at[pl.ds(sid * EW, EW)],
                        src_v.at[pl.ds(0, EW)])
        pltpu.sync_copy(src_hbm.at[pl.ds((NS + sid) * EW, EW)],
                        src_v.at[pl.ds(EW, EW)])
        pltpu.sync_copy(dst_hbm.at[pl.ds(sid * EW, EW)],
                        dst_v.at[pl.ds(0, EW)])
        pltpu.sync_copy(dst_hbm.at[pl.ds((NS + sid) * EW, EW)],
                        dst_v.at[pl.ds(EW, EW)])
        s = sbuf[...][0]

        def scan(i, bcnt):
            for u in range(5):
                off = (i * 5 + u) * 16
                d = dst_v[pl.ds(off, 16)]
                fl = plsc.load_gather(flag_v, [d])
                m = fl > 0
                mi = m.astype(i32)
                pos = bcnt + plsc.cumsum(mi) - mi
                plsc.store_scatter(beidx_v, [pos], lane + off, mask=m)
                bcnt = bcnt + plsc.all_reduce_population_count(m)[0]
            return bcnt

        assert NEV2 % 5 == 0
        bcnt = lax.fori_loop(0, NEV2 // 5, scan, jnp.asarray(0, i32))
        plsc.subcore_barrier()

        nrounds = (s + WIN - 1) // WIN

        def round_body(r, _):
            lo = r * WIN + cid * HW

            def zr(kk, __):
                pltpu.sync_copy(zrows_v.at[pl.ds(0, 32)],
                                acc_sh.at[pl.ds(sid * SWZ + kk * 32, 32)])
                return 0
            lax.fori_loop(0, SWZ // 32, zr, 0)
            plsc.subcore_barrier()

            def process_chunk(svec, w, cvec, valid):
                valid = jnp.logical_and(
                    valid, jnp.logical_and(cvec >= lo, cvec < lo + HW))
                w = jnp.where(valid, w, 0.0)
                gibuf[...] = svec
                sibuf[...] = jnp.where(valid, cvec - lo, 0)
                pltpu.async_copy(obs_hbm.at[gibuf], rows_v, sem).wait()
                for l in range(16):
                    wv = jnp.full((16,), w[l])
                    for c in range(CV):
                        rows_v[l, pl.ds(c * 16, 16)] = (
                            rows_v[l, pl.ds(c * 16, 16)] * wv
                        )
                pltpu.sync_copy(rows_v, acc_sh.at[sibuf], add=True)

            def flush(j, __):
                off = j * 16
                ev = beidx_v[pl.ds(off, 16)]
                valid = lane < (bcnt - off)
                ev = jnp.where(valid, ev, 0)
                sv = plsc.load_gather(src_v, [ev])
                dv = plsc.load_gather(dst_v, [ev])
                sv = jnp.where(valid, sv, 0)
                dv = jnp.where(valid, dv, 0)
                w = (plsc.load_gather(dinv_v, [sv])
                     * plsc.load_gather(dinv_v, [dv]))
                cvec = plsc.load_gather(comp_v, [dv])
                process_chunk(sv, w, cvec, valid)
                return 0
            lax.fori_loop(0, (bcnt + 15) // 16, flush, 0)

            def selfb(j, __):
                vv = lane + sid * SW2 + j * 16
                f = flag_v[pl.ds(sid * SW2 + j * 16, 16)]
                m = f > 0
                pc = plsc.all_reduce_population_count(m)[0]

                def hit(u):
                    dv_ = plsc.load_gather(dinv_v, [vv])
                    w = dv_ * dv_
                    cvec = plsc.load_gather(comp_v, [vv])
                    process_chunk(vv, w, cvec, m)
                    return 0
                return lax.cond(pc > 0, hit, lambda u: 0, 0)
            lax.fori_loop(0, SW2 // 16, selfb, 0)

            plsc.subcore_barrier()

            @pl.when(sid == 0)
            def _():
                sc = jnp.clip(s - lo, 0, HW)

                def outw(kk, __):
                    pltpu.sync_copy(acc_sh.at[pl.ds(kk * ZR, ZR)],
                                    a_hbm.at[pl.ds(lo + kk * ZR, ZR)])
                    return 0
                lax.fori_loop(0, (sc + ZR - 1) // ZR, outw, 0)
            plsc.subcore_barrier()
            return 0

        lax.fori_loop(0, nrounds, round_body, 0)

    return k


def _p5_matmul(NP2, DIN, H):
    BR = 256

    def body(s_ref, a_ref, w1_ref, b1_ref, h_ref, av, hv, sem0, sem1):
        s = s_ref[0]
        nb = (s + BR - 1) // BR

        def blk(kk, _):
            cp0 = pltpu.make_async_copy(a_ref.at[pl.ds(kk * BR, BR)], av, sem0)
            cp0.start()
            cp0.wait()
            h = jnp.maximum(
                jnp.dot(av[...], w1_ref[...], preferred_element_type=f32,
                        precision=lax.Precision.HIGHEST)
                + b1_ref[...], 0.0)
            hv[...] = jnp.concatenate([h, jnp.zeros_like(h)], axis=1)
            cph = pltpu.make_async_copy(
                hv, h_ref.at[pl.ds(kk * BR, BR)], sem1)
            cph.start()
            cph.wait()
            return 0
        lax.fori_loop(0, nb, blk, 0)

    return pl.pallas_call(
        body,
        in_specs=[
            pl.BlockSpec(memory_space=pltpu.SMEM),
            pl.BlockSpec(memory_space=pltpu.HBM),
            pl.BlockSpec(memory_space=pltpu.VMEM),
            pl.BlockSpec(memory_space=pltpu.VMEM),
        ],
        out_specs=pl.BlockSpec(memory_space=pltpu.HBM),
        out_shape=jax.ShapeDtypeStruct((NP2, 2 * H), f32),
        scratch_shapes=[
            pltpu.VMEM((BR, DIN), f32),
            pltpu.VMEM((BR, 2 * H), f32),
            pltpu.SemaphoreType.DMA,
            pltpu.SemaphoreType.DMA,
        ],
    )


def _p6_layer2(NP2, EW, N, H):
    CV = H // 16

    @functools.partial(
        pl.kernel,
        mesh=_mesh(),
        compiler_params=_SC_PARAMS,
        out_type=jax.ShapeDtypeStruct((NC * H,), f32),
        scratch_types=[
            pltpu.VMEM((NP2,), f32),
            pltpu.VMEM((NP2,), i32),
            pltpu.VMEM((16,), i32),
            pltpu.VMEM((16,), i32),
            pltpu.VMEM((16,), i32),
            pltpu.VMEM((16, 2 * H), f32),
            pltpu.VMEM((H,), f32),
            pltpu.VMEM((NS * H,), f32),
            pltpu.VMEM_SHARED((NS * H,), f32),
            pltpu.SemaphoreType.DMA,
        ],
    )
    def k(h_hbm, sel_hbm, cnt_hbm, dinv_hbm, comp_hbm, zpart_hbm,
          dinv_v, comp_v, cbuf, selb, gibuf, hrows_v, zacc_v, tmpall_v,
          zslots, sem):
        cid = lax.axis_index("c")
        sid = lax.axis_index("s")
        wid = cid * NS + sid
        lane = lax.iota(i32, 16)
        pltpu.sync_copy(cnt_hbm.at[pl.ds(wid * 16, 16)], cbuf)
        cw = cbuf[...][0]
        _zero_1d(zacc_v, CV, f32)

        @pl.when(jnp.logical_or(cw > 0, wid == 0))
        def _():
            pltpu.sync_copy(dinv_hbm, dinv_v)
            pltpu.sync_copy(comp_hbm, comp_v)
            dinvt = dinv_v[pl.ds(N - 16, 16)][15]

            def accum(w, rows16):
                for l in range(16):
                    wv = jnp.full((16,), w[l])
                    for c in range(CV):
                        zacc_v[pl.ds(c * 16, 16)] = (
                            zacc_v[pl.ds(c * 16, 16)]
                            + rows16[l, pl.ds(c * 16, 16)] * wv
                        )

            def chunk(j, _):
                pltpu.sync_copy(sel_hbm.at[pl.ds(wid * EW + j * 16, 16)],
                                selb)
                vv = selb[...]
                valid = lane < (cw - j * 16)
                vv = jnp.where(valid, vv, 0)
                w = jnp.where(valid,
                              plsc.load_gather(dinv_v, [vv]) * dinvt, 0.0)
                gibuf[...] = jnp.where(valid,
                                       plsc.load_gather(comp_v, [vv]), 0)
                pltpu.async_copy(h_hbm.at[gibuf], hrows_v, sem).wait()
                accum(w, hrows_v)
                return 0
            lax.fori_loop(0, (cw + 15) // 16, chunk, 0)

            @pl.when(wid == 0)
            def _():
                gibuf[...] = jnp.full((16,), comp_v[pl.ds(N - 16, 16)][15])
                pltpu.async_copy(h_hbm.at[gibuf], hrows_v, sem).wait()
                w = jnp.where(lane == 0, dinvt * dinvt, 0.0)
                accum(w, hrows_v)

        pltpu.sync_copy(zacc_v, zslots.at[pl.ds(sid * H, H)])
        plsc.subcore_barrier()

        @pl.when(sid == 0)
        def _():
            pltpu.sync_copy(zslots, tmpall_v)
            _zero_1d(zacc_v, CV, f32)
            for j in range(NS):
                for c in range(CV):
                    zacc_v[pl.ds(c * 16, 16)] = (
                        zacc_v[pl.ds(c * 16, 16)]
                        + tmpall_v[pl.ds(j * H + c * 16, 16)]
                    )
            pltpu.sync_copy(zacc_v, zpart_hbm.at[pl.ds(cid * H, H)])

    return k


def _p7_head(H, DOUT):
    def body(zp_ref, w2_ref, b2_ref, w3_ref, b3_ref, out_ref):
        z = zp_ref[0:1, :] + zp_ref[1:2, :]
        z8 = jnp.broadcast_to(z, (8, z.shape[1]))
        h2 = jnp.maximum(
            jnp.dot(z8, w2_ref[...], preferred_element_type=f32,
                    precision=lax.Precision.HIGHEST) + b2_ref[...],
            0.0)
        out_ref[...] = (
            jnp.dot(h2, w3_ref[...], preferred_element_type=f32,
                    precision=lax.Precision.HIGHEST) + b3_ref[...]
        )

    return pl.pallas_call(
        body,
        out_shape=jax.ShapeDtypeStruct((8, DOUT), f32),
    )


def kernel(obs, edge_index, W1, b1, W2, b2, W3, b3):
    N, DIN = obs.shape
    E = edge_index.shape[1]
    H = W1.shape[1]
    DOUT = W3.shape[1]
    assert E % NW == 0
    EW = E // NW
    assert EW % 16 == 0
    NP2 = ((N + 1279) // 1280) * 1280
    zer = jnp.zeros((64, DIN), f32)

    esrc = edge_index[0]
    edst = edge_index[1]
    hist, sel, cnt = _p1_hist_select(NP2, E, EW, N)(esrc, edst)
    flag = _p3a_flag(NP2, EW, N)(sel, cnt)
    dinv, comp, s = _p2_dinv_comp(NP2)(hist.reshape(NW, NP2 // 128, 128),
                                       flag.reshape(NP2 // 128, 128))
    dinv = dinv.reshape(NP2)
    comp = comp.reshape(NP2)
    s16 = s.reshape(-1)[:16]
    a = _p4_aggregate(NP2, E, EW, N, DIN)(
        esrc, edst, obs, dinv, flag, comp, s16, zer)
    h = _p5_matmul(NP2, DIN, H)(s16, a, W1, b1.reshape(1, H))
    zpart = _p6_layer2(NP2, EW, N, H)(h, sel, cnt, dinv, comp)
    out = _p7_head(H, DOUT)(zpart.reshape(NC, H), W2, b2.reshape(1, H), W3,
                            b3.reshape(1, DOUT))
    return out[0]
